# baseline (speedup 1.0000x reference)
import jax
import jax.numpy as jnp
from jax import lax
from jax.experimental import pallas as pl
from jax.experimental.pallas import tpu as pltpu
from jax.experimental.pallas import tpu_sc as plsc

H = 1080
W = 1920
HW = H * W
EPS = 1e-8
P = 1_000_000

NUM_TILES = 16
CHUNK = 2048
NVEC = CHUNK // 16
NB = CHUNK // 128
PER_TILE = 63488
NCHUNKS = PER_TILE // CHUNK
PPAD = PER_TILE * NUM_TILES

QSIZE = HW // 6
QROWS = 360448
ZROWS = QROWS // NUM_TILES
NPIX = QSIZE // NUM_TILES
NCH = 1200

TWO_PI_INV = float(1.0 / (2.0 * 3.141592653589793))


def _rsqrt(x):
    i = plsc.bitcast(x, jnp.int32)
    i = jnp.int32(0x5F3759DF) - (i >> 1)
    y = plsc.bitcast(i, jnp.float32)
    for _ in range(3):
        y = y * (1.5 - 0.5 * x * y * y)
    return y


def _body(means, covs, depth, colors, out,
          mbuf, cbuf, dbuf, nbuf, accr, accg, accb, accw, idx2,
          vr, vg, vb, vw, rr, rg, rb, rw, obr, obg, obb, zbuf):
    core = lax.axis_index("c")
    tile = lax.axis_index("s")

    lanes = lax.iota(jnp.int32, 16)
    zf = jnp.zeros((16,), jnp.float32)

    def zinit(i, carry):
        zbuf[pl.ds(i * 16, 16)] = zf
        return carry

    lax.fori_loop(0, 128, zinit, jnp.int32(0))

    for p in range(3):
        qbase = (core * 3 + p) * QSIZE

        zrow0 = pl.multiple_of(tile * ZROWS, 8)

        def zero(z, carry):
            zo = pl.multiple_of(zrow0 + z * 2048, 8)
            pltpu.sync_copy(zbuf, accr.at[pl.ds(zo, 2048)])
            pltpu.sync_copy(zbuf, accg.at[pl.ds(zo, 2048)])
            pltpu.sync_copy(zbuf, accb.at[pl.ds(zo, 2048)])
            pltpu.sync_copy(zbuf, accw.at[pl.ds(zo, 2048)])
            return carry

        lax.fori_loop(0, ZROWS // 2048, zero, jnp.int32(0))
        plsc.subcore_barrier()

        gtile0 = pl.multiple_of(tile * PER_TILE, 8)

        def chunk_body(ch, ccarry):
            g0 = pl.multiple_of(gtile0 + ch * CHUNK, 8)
            pltpu.sync_copy(means.at[pl.ds(g0 * 2, CHUNK * 2)], mbuf)
            pltpu.sync_copy(covs.at[pl.ds(g0 * 3, CHUNK * 3)], cbuf)
            pltpu.sync_copy(depth.at[pl.ds(g0, CHUNK)], dbuf)
            pltpu.sync_copy(colors.at[pl.ds(g0 * 3, CHUNK * 3)], nbuf)

            def compute(i, cnt):
                r2 = i * 32 + lanes * 2
                r3 = i * 48 + lanes * 3
                x = plsc.load_gather(mbuf, [r2])
                y = plsc.load_gather(mbuf, [r2 + 1])
                ca = plsc.load_gather(cbuf, [r3])
                cb = plsc.load_gather(cbuf, [r3 + 1])
                cc = plsc.load_gather(cbuf, [r3 + 2])
                dp = dbuf[pl.ds(i * 16, 16)]
                cr = plsc.load_gather(nbuf, [r3])
                cg = plsc.load_gather(nbuf, [r3 + 1])
                cbl = plsc.load_gather(nbuf, [r3 + 2])

                px = (x * jnp.float32(W)).astype(jnp.int32)
                px = jnp.minimum(jnp.maximum(px, 0), W - 1)
                py = (y * jnp.float32(H)).astype(jnp.int32)
                py = jnp.minimum(jnp.maximum(py, 0), H - 1)
                pid = py * W + px

                det = jnp.maximum(ca * cc - cb * cb, jnp.float32(EPS))
                wgt = _rsqrt(det) * jnp.float32(TWO_PI_INV) * jnp.exp(-dp)

                local = pid - qbase
                inq = (local >= 0) & (local < QSIZE)
                inq_i = inq.astype(jnp.int32)
                pos = cnt + jnp.cumsum(inq_i) - 1
                plsc.store_scatter(idx2, [pos >> 7, pos & 127], local, mask=inq)
                plsc.store_scatter(vr, [pos], wgt * cr, mask=inq)
                plsc.store_scatter(vg, [pos], wgt * cg, mask=inq)
                plsc.store_scatter(vb, [pos], wgt * cbl, mask=inq)
                plsc.store_scatter(vw, [pos], wgt, mask=inq)
                return cnt + jnp.sum(inq_i)

            cnt = lax.fori_loop(0, NVEC, compute, jnp.int32(0))

            tb0 = lax.bitwise_and(cnt, jnp.int32(-128))
            tend = lax.bitwise_and(cnt + 127, jnp.int32(-128))
            for v in range(8):
                tpos = tb0 + v * 16 + lanes
                tmask = (tpos >= cnt) & (tpos < tend)
                plsc.store_scatter(idx2, [tpos >> 7, tpos & 127],
                                   jnp.full((16,), QSIZE, jnp.int32), mask=tmask)

            nbatch = (cnt + 127) >> 7

            def scat(j, carry):
                @pl.when(j < nbatch)
                def _():
                    sl = pl.ds(j * 128, 128)
                    idx = idx2.at[j]
                    pltpu.sync_copy(vr.at[sl], accr.at[idx], add=True)
                    pltpu.sync_copy(vg.at[sl], accg.at[idx], add=True)
                    pltpu.sync_copy(vb.at[sl], accb.at[idx], add=True)
                    pltpu.sync_copy(vw.at[sl], accw.at[idx], add=True)
                return carry

            lax.fori_loop(0, NB, scat, jnp.int32(0))
            return ccarry

        lax.fori_loop(0, NCHUNKS, chunk_body, jnp.int32(0))

        plsc.subcore_barrier()

        prow0 = pl.multiple_of(tile * NPIX, 8)

        def norm_chunk(nc, ncarry):
            r0 = pl.multiple_of(prow0 + nc * NCH, 8)
            pltpu.sync_copy(accr.at[pl.ds(r0, NCH)], rr)
            pltpu.sync_copy(accg.at[pl.ds(r0, NCH)], rg)
            pltpu.sync_copy(accb.at[pl.ds(r0, NCH)], rb)
            pltpu.sync_copy(accw.at[pl.ds(r0, NCH)], rw)

            def norm(i, carry):
                sl = pl.ds(i * 16, 16)
                d = rw[sl] + jnp.float32(EPS)
                obr[sl] = rr[sl] / d
                obg[sl] = rg[sl] / d
                obb[sl] = rb[sl] / d
                return carry

            lax.fori_loop(0, NCH // 16, norm, jnp.int32(0))
            gpix = pl.multiple_of(qbase + r0, 8)
            for c, ob in enumerate((obr, obg, obb)):
                pltpu.sync_copy(ob, out.at[pl.ds(c * HW + gpix, NCH)])
            return ncarry

        lax.fori_loop(0, NPIX // NCH, norm_chunk, jnp.int32(0))

        plsc.subcore_barrier()


def kernel(means_2d, covs_2d, depth_features, color_features, height, width):
    pad = PPAD - P
    means_p = jnp.pad(means_2d, ((0, pad), (0, 0))).reshape(PPAD * 2)
    covs_p = jnp.pad(covs_2d, ((0, pad), (0, 0)), constant_values=1.0).reshape(PPAD * 3)
    depth_p = jnp.pad(depth_features.reshape(P), (0, pad), constant_values=60.0)
    colors_p = jnp.pad(color_features, ((0, pad), (0, 0))).reshape(PPAD * 3)

    mesh = plsc.VectorSubcoreMesh(core_axis_name="c", subcore_axis_name="s")
    run = pl.kernel(
        _body,
        out_type=jax.ShapeDtypeStruct((3 * HW,), jnp.float32),
        mesh=mesh,
        compiler_params=pltpu.CompilerParams(needs_layout_passes=False),
        scratch_types=[
            pltpu.VMEM((CHUNK * 2,), jnp.float32),
            pltpu.VMEM((CHUNK * 3,), jnp.float32),
            pltpu.VMEM((CHUNK,), jnp.float32),
            pltpu.VMEM((CHUNK * 3,), jnp.float32),
            pltpu.VMEM_SHARED((QROWS,), jnp.float32),
            pltpu.VMEM_SHARED((QROWS,), jnp.float32),
            pltpu.VMEM_SHARED((QROWS,), jnp.float32),
            pltpu.VMEM_SHARED((QROWS,), jnp.float32),
            pltpu.VMEM((NB, 128), jnp.int32),
            pltpu.VMEM((CHUNK,), jnp.float32),
            pltpu.VMEM((CHUNK,), jnp.float32),
            pltpu.VMEM((CHUNK,), jnp.float32),
            pltpu.VMEM((CHUNK,), jnp.float32),
            pltpu.VMEM((NCH,), jnp.float32),
            pltpu.VMEM((NCH,), jnp.float32),
            pltpu.VMEM((NCH,), jnp.float32),
            pltpu.VMEM((NCH,), jnp.float32),
            pltpu.VMEM((NCH,), jnp.float32),
            pltpu.VMEM((NCH,), jnp.float32),
            pltpu.VMEM((NCH,), jnp.float32),
            pltpu.VMEM((2048,), jnp.float32),
        ],
    )
    out = run(means_p, covs_p, depth_p, colors_p)
    return out.reshape(3, H, W)


# speedup vs baseline: 1.0029x; 1.0029x over previous
import jax
import jax.numpy as jnp
from jax import lax
from jax.experimental import pallas as pl
from jax.experimental.pallas import tpu as pltpu
from jax.experimental.pallas import tpu_sc as plsc

H = 1080
W = 1920
HW = H * W
EPS = 1e-8
P = 1_000_000

NUM_TILES = 16
CHUNK = 1024
NVEC = CHUNK // 16
NB = CHUNK // 128
PER_TILE = 65536
NCHUNKS = PER_TILE // CHUNK
NPAIR = NCHUNKS // 2
PPAD = PER_TILE * NUM_TILES

QSIZE = HW // 6
QROWS = 345728
ZROWS = QROWS // NUM_TILES
ZTAIL = ZROWS - 10 * 2048
NPIX = QSIZE // NUM_TILES
NCH = 1200

TWO_PI_INV = float(1.0 / (2.0 * 3.141592653589793))


def _rsqrt(x):
    i = plsc.bitcast(x, jnp.int32)
    i = jnp.int32(0x5F3759DF) - (i >> 1)
    y = plsc.bitcast(i, jnp.float32)
    for _ in range(3):
        y = y * (1.5 - 0.5 * x * y * y)
    return y


def _body(means, covs, depth, colors, out,
          mA, cA, dA, nA, mB, cB, dB, nB,
          accr, accg, accb, accw,
          idxA, vrA, vgA, vbA, vwA,
          idxB, vrB, vgB, vbB, vwB,
          rr, rg, rb, rw, obr, obg, obb, zbuf,
          semA, semB, semSA, semSB, semZ, semN, semO):
    core = lax.axis_index("c")
    tile = lax.axis_index("s")

    lanes = lax.iota(jnp.int32, 16)
    zf = jnp.zeros((16,), jnp.float32)

    def zinit(i, carry):
        zbuf[pl.ds(i * 16, 16)] = zf
        return carry

    lax.fori_loop(0, 128, zinit, jnp.int32(0))

    gtile0 = pl.multiple_of(tile * PER_TILE, 8)
    accs = (accr, accg, accb, accw)

    def in_slices(g0):
        return (means.at[pl.ds(pl.multiple_of(g0 * 2, 16), CHUNK * 2)],
                covs.at[pl.ds(pl.multiple_of(g0 * 3, 8), CHUNK * 3)],
                depth.at[pl.ds(g0, CHUNK)],
                colors.at[pl.ds(pl.multiple_of(g0 * 3, 8), CHUNK * 3)])

    def start_in(g0, bufs, sem):
        for src, dst in zip(in_slices(g0), bufs):
            pltpu.async_copy(src, dst, sem)

    def wait_in(g0, bufs, sem):
        for src, dst in zip(in_slices(g0), bufs):
            pltpu.make_async_copy(src, dst, sem).wait()

    def make_compute(bufs, idx2, vr, vg, vb, vw, qbase):
        mb, cb2, db, nb2 = bufs

        def compute(i, cnt):
            r2 = i * 32 + lanes * 2
            r3 = i * 48 + lanes * 3
            x = plsc.load_gather(mb, [r2])
            y = plsc.load_gather(mb, [r2 + 1])
            ca = plsc.load_gather(cb2, [r3])
            cb = plsc.load_gather(cb2, [r3 + 1])
            cc = plsc.load_gather(cb2, [r3 + 2])
            dp = db[pl.ds(i * 16, 16)]
            cr = plsc.load_gather(nb2, [r3])
            cg = plsc.load_gather(nb2, [r3 + 1])
            cbl = plsc.load_gather(nb2, [r3 + 2])

            px = (x * jnp.float32(W)).astype(jnp.int32)
            px = jnp.minimum(jnp.maximum(px, 0), W - 1)
            py = (y * jnp.float32(H)).astype(jnp.int32)
            py = jnp.minimum(jnp.maximum(py, 0), H - 1)
            pid = py * W + px

            det = jnp.maximum(ca * cc - cb * cb, jnp.float32(EPS))
            wgt = _rsqrt(det) * jnp.float32(TWO_PI_INV) * jnp.exp(-dp)

            local = pid - qbase
            inq = (local >= 0) & (local < QSIZE)
            inq_i = inq.astype(jnp.int32)
            pos = cnt + jnp.cumsum(inq_i) - 1
            plsc.store_scatter(idx2, [pos >> 7, pos & 127], local, mask=inq)
            plsc.store_scatter(vr, [pos], wgt * cr, mask=inq)
            plsc.store_scatter(vg, [pos], wgt * cg, mask=inq)
            plsc.store_scatter(vb, [pos], wgt * cbl, mask=inq)
            plsc.store_scatter(vw, [pos], wgt, mask=inq)
            return cnt + jnp.sum(inq_i)

        cnt = lax.fori_loop(0, NVEC, compute, jnp.int32(0))

        tb0 = lax.bitwise_and(cnt, jnp.int32(-128))
        tend = lax.bitwise_and(cnt + 127, jnp.int32(-128))
        for v in range(8):
            tpos = tb0 + v * 16 + lanes
            tmask = (tpos >= cnt) & (tpos < tend)
            plsc.store_scatter(idx2, [tpos >> 7, tpos & 127],
                               jnp.full((16,), QSIZE, jnp.int32), mask=tmask)
        return (cnt + 127) >> 7

    def fire_scatter(nbatch, idx2, vr, vg, vb, vw, sem):
        def fire(j, carry):
            @pl.when(j < nbatch)
            def _():
                sl = pl.ds(j * 128, 128)
                idx = idx2.at[j]
                for v, acc in zip((vr, vg, vb, vw), accs):
                    pltpu.async_copy(v.at[sl], acc.at[idx], sem)
            return carry

        lax.fori_loop(0, NB, fire, jnp.int32(0))

    def sync_scatter(nbatch, idx2, vr, vg, vb, vw):
        def scat(j, carry):
            @pl.when(j < nbatch)
            def _():
                sl = pl.ds(j * 128, 128)
                idx = idx2.at[j]
                for v, acc in zip((vr, vg, vb, vw), accs):
                    pltpu.sync_copy(v.at[sl], acc.at[idx], add=True)
            return carry

        lax.fori_loop(0, NB, scat, jnp.int32(0))

    for p in range(3):
        qbase = (core * 3 + p) * QSIZE

        zrow0 = pl.multiple_of(tile * ZROWS, 8)

        def zfire(z, carry):
            zo = pl.multiple_of(zrow0 + z * 2048, 8)
            for acc in accs:
                pltpu.async_copy(zbuf, acc.at[pl.ds(zo, 2048)], semZ)
            return carry

        def zdrain(z, carry):
            zo = pl.multiple_of(zrow0 + z * 2048, 8)
            for acc in accs:
                pltpu.make_async_copy(zbuf, acc.at[pl.ds(zo, 2048)], semZ).wait()
            return carry

        lax.fori_loop(0, 10, zfire, jnp.int32(0))
        ztail0 = pl.multiple_of(zrow0 + 10 * 2048, 8)
        for acc in accs:
            pltpu.async_copy(zbuf.at[pl.ds(0, ZTAIL)],
                             acc.at[pl.ds(ztail0, ZTAIL)], semZ)
        lax.fori_loop(0, 10, zdrain, jnp.int32(0))
        for acc in accs:
            pltpu.make_async_copy(zbuf.at[pl.ds(0, ZTAIL)],
                                  acc.at[pl.ds(ztail0, ZTAIL)], semZ).wait()
        plsc.subcore_barrier()

        bufsA = (mA, cA, dA, nA)
        bufsB = (mB, cB, dB, nB)
        start_in(gtile0, bufsA, semA)

        def pair(i, nbB_prev):
            g0 = pl.multiple_of(gtile0 + (2 * i) * CHUNK, 8)
            g1 = pl.multiple_of(g0 + CHUNK, 8)

            wait_in(g0, bufsA, semA)
            start_in(g1, bufsB, semB)
            nbA = make_compute(bufsA, idxA, vrA, vgA, vbA, vwA, qbase)
            sync_scatter(nbA, idxA, vrA, vgA, vbA, vwA)

            wait_in(g1, bufsB, semB)

            @pl.when(i < NPAIR - 1)
            def _():
                g2 = pl.multiple_of(g1 + CHUNK, 8)
                start_in(g2, bufsA, semA)

            nbB = make_compute(bufsB, idxB, vrB, vgB, vbB, vwB, qbase)
            sync_scatter(nbB, idxB, vrB, vgB, vbB, vwB)
            return nbB_prev

        lax.fori_loop(0, NPAIR, pair, jnp.int32(0))

        plsc.subcore_barrier()

        prow0 = pl.multiple_of(tile * NPIX, 8)

        def norm_chunk(nc, ncarry):
            r0 = pl.multiple_of(prow0 + nc * NCH, 8)
            for acc, rbuf in zip(accs, (rr, rg, rb, rw)):
                pltpu.async_copy(acc.at[pl.ds(r0, NCH)], rbuf, semN)
            for acc, rbuf in zip(accs, (rr, rg, rb, rw)):
                pltpu.make_async_copy(acc.at[pl.ds(r0, NCH)], rbuf, semN).wait()

            gpix_prev = pl.multiple_of(qbase + r0 - NCH, 8)

            @pl.when(nc > 0)
            def _():
                for c, ob in enumerate((obr, obg, obb)):
                    pltpu.make_async_copy(
                        ob, out.at[pl.ds(c * HW + gpix_prev, NCH)], semO).wait()

            def norm(i, carry):
                sl = pl.ds(i * 16, 16)
                d = rw[sl] + jnp.float32(EPS)
                obr[sl] = rr[sl] / d
                obg[sl] = rg[sl] / d
                obb[sl] = rb[sl] / d
                return carry

            lax.fori_loop(0, NCH // 16, norm, jnp.int32(0))
            gpix = pl.multiple_of(qbase + r0, 8)
            for c, ob in enumerate((obr, obg, obb)):
                pltpu.async_copy(ob, out.at[pl.ds(c * HW + gpix, NCH)], semO)
            return ncarry

        lax.fori_loop(0, NPIX // NCH, norm_chunk, jnp.int32(0))
        gpix_last = pl.multiple_of(qbase + prow0 + NPIX - NCH, 8)
        for c, ob in enumerate((obr, obg, obb)):
            pltpu.make_async_copy(
                ob, out.at[pl.ds(c * HW + gpix_last, NCH)], semO).wait()

        plsc.subcore_barrier()


def kernel(means_2d, covs_2d, depth_features, color_features, height, width):
    pad = PPAD - P
    means_p = jnp.pad(means_2d, ((0, pad), (0, 0))).reshape(PPAD * 2)
    covs_p = jnp.pad(covs_2d, ((0, pad), (0, 0)), constant_values=1.0).reshape(PPAD * 3)
    depth_p = jnp.pad(depth_features.reshape(P), (0, pad), constant_values=60.0)
    colors_p = jnp.pad(color_features, ((0, pad), (0, 0))).reshape(PPAD * 3)

    mesh = plsc.VectorSubcoreMesh(core_axis_name="c", subcore_axis_name="s")
    run = pl.kernel(
        _body,
        out_type=jax.ShapeDtypeStruct((3 * HW,), jnp.float32),
        mesh=mesh,
        compiler_params=pltpu.CompilerParams(needs_layout_passes=False),
        scratch_types=[
            pltpu.VMEM((CHUNK * 2,), jnp.float32),
            pltpu.VMEM((CHUNK * 3,), jnp.float32),
            pltpu.VMEM((CHUNK,), jnp.float32),
            pltpu.VMEM((CHUNK * 3,), jnp.float32),
            pltpu.VMEM((CHUNK * 2,), jnp.float32),
            pltpu.VMEM((CHUNK * 3,), jnp.float32),
            pltpu.VMEM((CHUNK,), jnp.float32),
            pltpu.VMEM((CHUNK * 3,), jnp.float32),
            pltpu.VMEM_SHARED((QROWS,), jnp.float32),
            pltpu.VMEM_SHARED((QROWS,), jnp.float32),
            pltpu.VMEM_SHARED((QROWS,), jnp.float32),
            pltpu.VMEM_SHARED((QROWS,), jnp.float32),
            pltpu.VMEM((NB, 128), jnp.int32),
            pltpu.VMEM((CHUNK,), jnp.float32),
            pltpu.VMEM((CHUNK,), jnp.float32),
            pltpu.VMEM((CHUNK,), jnp.float32),
            pltpu.VMEM((CHUNK,), jnp.float32),
            pltpu.VMEM((NB, 128), jnp.int32),
            pltpu.VMEM((CHUNK,), jnp.float32),
            pltpu.VMEM((CHUNK,), jnp.float32),
            pltpu.VMEM((CHUNK,), jnp.float32),
            pltpu.VMEM((CHUNK,), jnp.float32),
            pltpu.VMEM((NCH,), jnp.float32),
            pltpu.VMEM((NCH,), jnp.float32),
            pltpu.VMEM((NCH,), jnp.float32),
            pltpu.VMEM((NCH,), jnp.float32),
            pltpu.VMEM((NCH,), jnp.float32),
            pltpu.VMEM((NCH,), jnp.float32),
            pltpu.VMEM((NCH,), jnp.float32),
            pltpu.VMEM((2048,), jnp.float32),
            pltpu.SemaphoreType.DMA,
            pltpu.SemaphoreType.DMA,
            pltpu.SemaphoreType.DMA,
            pltpu.SemaphoreType.DMA,
            pltpu.SemaphoreType.DMA,
            pltpu.SemaphoreType.DMA,
            pltpu.SemaphoreType.DMA,
        ],
    )
    out = run(means_p, covs_p, depth_p, colors_p)
    return out.reshape(3, H, W)


# speedup vs baseline: 8.1394x; 8.1155x over previous
import jax
import jax.numpy as jnp
from jax import lax
from jax.experimental import pallas as pl
from jax.experimental.pallas import tpu as pltpu
from jax.experimental.pallas import tpu_sc as plsc

H = 1080
W = 1920
HW = H * W
EPS = 1e-8
P = 1_000_000

NUM_TILES = 16
CHUNK = 1024
NVEC = CHUNK // 16
NB = CHUNK // 128
PER_TILE = 65536
NCHUNKS = PER_TILE // CHUNK
NPAIR = NCHUNKS // 2
PPAD = PER_TILE * NUM_TILES

QSIZE = HW // 6
QROWS = 345728
ZROWS = QROWS // NUM_TILES
ZTAIL = ZROWS - 10 * 2048
NPIX = QSIZE // NUM_TILES
NCH = 1200

TWO_PI_INV = float(1.0 / (2.0 * 3.141592653589793))


def _rsqrt(x):
    i = plsc.bitcast(x, jnp.int32)
    i = jnp.int32(0x5F3759DF) - (i >> 1)
    y = plsc.bitcast(i, jnp.float32)
    for _ in range(3):
        y = y * (1.5 - 0.5 * x * y * y)
    return y


def _body(xs, ys, ca_, cb_, cc_, dp_, cr_, cg_, cbl_, out,
          iA0, iA1, iA2, iA3, iA4, iA5, iA6, iA7, iA8,
          iB0, iB1, iB2, iB3, iB4, iB5, iB6, iB7, iB8,
          accr, accg, accb, accw,
          idxA, vrA, vgA, vbA, vwA,
          idxB, vrB, vgB, vbB, vwB,
          rr, rg, rb, rw, obr, obg, obb, zbuf,
          semA, semB, semSA, semSB, semZ, semN, semO):
    core = lax.axis_index("c")
    tile = lax.axis_index("s")

    lanes = lax.iota(jnp.int32, 16)
    zf = jnp.zeros((16,), jnp.float32)

    def zinit(i, carry):
        zbuf[pl.ds(i * 16, 16)] = zf
        return carry

    lax.fori_loop(0, 128, zinit, jnp.int32(0))

    gtile0 = pl.multiple_of(tile * PER_TILE, 8)
    accs = (accr, accg, accb, accw)
    bufsA = (iA0, iA1, iA2, iA3, iA4, iA5, iA6, iA7, iA8)
    bufsB = (iB0, iB1, iB2, iB3, iB4, iB5, iB6, iB7, iB8)

    planes = (xs, ys, ca_, cb_, cc_, dp_, cr_, cg_, cbl_)

    def in_slices(g0):
        return tuple(pln.at[pl.ds(g0, CHUNK)] for pln in planes)

    def start_in(g0, bufs, sem):
        for src, dst in zip(in_slices(g0), bufs):
            pltpu.async_copy(src, dst, sem)

    def wait_in(g0, bufs, sem):
        for src, dst in zip(in_slices(g0), bufs):
            pltpu.make_async_copy(src, dst, sem).wait()

    def make_compute(bufs, idx2, vr, vg, vb, vw, qbase):
        bx, by, bca, bcb, bcc, bdp, bcr, bcg, bcbl = bufs

        def compute(i, cnt):
            sl16 = pl.ds(i * 16, 16)
            x = bx[sl16]
            y = by[sl16]
            ca = bca[sl16]
            cb = bcb[sl16]
            cc = bcc[sl16]
            dp = bdp[sl16]
            cr = bcr[sl16]
            cg = bcg[sl16]
            cbl = bcbl[sl16]

            px = (x * jnp.float32(W)).astype(jnp.int32)
            px = jnp.minimum(jnp.maximum(px, 0), W - 1)
            py = (y * jnp.float32(H)).astype(jnp.int32)
            py = jnp.minimum(jnp.maximum(py, 0), H - 1)
            pid = py * W + px

            det = jnp.maximum(ca * cc - cb * cb, jnp.float32(EPS))
            wgt = _rsqrt(det) * jnp.float32(TWO_PI_INV) * jnp.exp(-dp)

            local = pid - qbase
            inq = (local >= 0) & (local < QSIZE)
            inq_i = inq.astype(jnp.int32)
            pos = cnt + jnp.cumsum(inq_i) - 1
            plsc.store_scatter(idx2, [pos >> 7, pos & 127], local, mask=inq)
            plsc.store_scatter(vr, [pos], wgt * cr, mask=inq)
            plsc.store_scatter(vg, [pos], wgt * cg, mask=inq)
            plsc.store_scatter(vb, [pos], wgt * cbl, mask=inq)
            plsc.store_scatter(vw, [pos], wgt, mask=inq)
            return cnt + jnp.sum(inq_i)

        cnt = lax.fori_loop(0, NVEC, compute, jnp.int32(0))

        tb0 = lax.bitwise_and(cnt, jnp.int32(-128))
        tend = lax.bitwise_and(cnt + 127, jnp.int32(-128))
        for v in range(8):
            tpos = tb0 + v * 16 + lanes
            tmask = (tpos >= cnt) & (tpos < tend)
            plsc.store_scatter(idx2, [tpos >> 7, tpos & 127],
                               jnp.full((16,), QSIZE, jnp.int32), mask=tmask)
        return (cnt + 127) >> 7

    def fire_scatter(nbatch, idx2, vr, vg, vb, vw, sem):
        def fire(j, carry):
            @pl.when(j < nbatch)
            def _():
                sl = pl.ds(j * 128, 128)
                idx = idx2.at[j]
                for v, acc in zip((vr, vg, vb, vw), accs):
                    pltpu.async_copy(v.at[sl], acc.at[idx], sem)
            return carry

        lax.fori_loop(0, NB, fire, jnp.int32(0))

    def sync_scatter(nbatch, idx2, vr, vg, vb, vw):
        def scat(j, carry):
            @pl.when(j < nbatch)
            def _():
                sl = pl.ds(j * 128, 128)
                idx = idx2.at[j]
                for v, acc in zip((vr, vg, vb, vw), accs):
                    pltpu.sync_copy(v.at[sl], acc.at[idx], add=True)
            return carry

        lax.fori_loop(0, NB, scat, jnp.int32(0))

    for p in range(3):
        qbase = (core * 3 + p) * QSIZE

        zrow0 = pl.multiple_of(tile * ZROWS, 8)

        def zfire(z, carry):
            zo = pl.multiple_of(zrow0 + z * 2048, 8)
            for acc in accs:
                pltpu.async_copy(zbuf, acc.at[pl.ds(zo, 2048)], semZ)
            return carry

        def zdrain(z, carry):
            zo = pl.multiple_of(zrow0 + z * 2048, 8)
            for acc in accs:
                pltpu.make_async_copy(zbuf, acc.at[pl.ds(zo, 2048)], semZ).wait()
            return carry

        lax.fori_loop(0, 10, zfire, jnp.int32(0))
        ztail0 = pl.multiple_of(zrow0 + 10 * 2048, 8)
        for acc in accs:
            pltpu.async_copy(zbuf.at[pl.ds(0, ZTAIL)],
                             acc.at[pl.ds(ztail0, ZTAIL)], semZ)
        lax.fori_loop(0, 10, zdrain, jnp.int32(0))
        for acc in accs:
            pltpu.make_async_copy(zbuf.at[pl.ds(0, ZTAIL)],
                                  acc.at[pl.ds(ztail0, ZTAIL)], semZ).wait()
        plsc.subcore_barrier()

        start_in(gtile0, bufsA, semA)

        def pair(i, nbB_prev):
            g0 = pl.multiple_of(gtile0 + (2 * i) * CHUNK, 8)
            g1 = pl.multiple_of(g0 + CHUNK, 8)

            wait_in(g0, bufsA, semA)
            start_in(g1, bufsB, semB)
            nbA = make_compute(bufsA, idxA, vrA, vgA, vbA, vwA, qbase)
            sync_scatter(nbA, idxA, vrA, vgA, vbA, vwA)

            wait_in(g1, bufsB, semB)

            @pl.when(i < NPAIR - 1)
            def _():
                g2 = pl.multiple_of(g1 + CHUNK, 8)
                start_in(g2, bufsA, semA)

            nbB = make_compute(bufsB, idxB, vrB, vgB, vbB, vwB, qbase)
            sync_scatter(nbB, idxB, vrB, vgB, vbB, vwB)
            return nbB_prev

        lax.fori_loop(0, NPAIR, pair, jnp.int32(0))

        plsc.subcore_barrier()

        prow0 = pl.multiple_of(tile * NPIX, 8)

        def norm_chunk(nc, ncarry):
            r0 = pl.multiple_of(prow0 + nc * NCH, 8)
            for acc, rbuf in zip(accs, (rr, rg, rb, rw)):
                pltpu.async_copy(acc.at[pl.ds(r0, NCH)], rbuf, semN)
            for acc, rbuf in zip(accs, (rr, rg, rb, rw)):
                pltpu.make_async_copy(acc.at[pl.ds(r0, NCH)], rbuf, semN).wait()

            gpix_prev = pl.multiple_of(qbase + r0 - NCH, 8)

            @pl.when(nc > 0)
            def _():
                for c, ob in enumerate((obr, obg, obb)):
                    pltpu.make_async_copy(
                        ob, out.at[pl.ds(c * HW + gpix_prev, NCH)], semO).wait()

            def norm(i, carry):
                sl = pl.ds(i * 16, 16)
                d = rw[sl] + jnp.float32(EPS)
                obr[sl] = rr[sl] / d
                obg[sl] = rg[sl] / d
                obb[sl] = rb[sl] / d
                return carry

            lax.fori_loop(0, NCH // 16, norm, jnp.int32(0))
            gpix = pl.multiple_of(qbase + r0, 8)
            for c, ob in enumerate((obr, obg, obb)):
                pltpu.async_copy(ob, out.at[pl.ds(c * HW + gpix, NCH)], semO)
            return ncarry

        lax.fori_loop(0, NPIX // NCH, norm_chunk, jnp.int32(0))
        gpix_last = pl.multiple_of(qbase + prow0 + NPIX - NCH, 8)
        for c, ob in enumerate((obr, obg, obb)):
            pltpu.make_async_copy(
                ob, out.at[pl.ds(c * HW + gpix_last, NCH)], semO).wait()

        plsc.subcore_barrier()


def kernel(means_2d, covs_2d, depth_features, color_features, height, width):
    pad = PPAD - P
    xs = jnp.pad(means_2d[:, 0], (0, pad))
    ys = jnp.pad(means_2d[:, 1], (0, pad))
    ca = jnp.pad(covs_2d[:, 0], (0, pad), constant_values=1.0)
    cb = jnp.pad(covs_2d[:, 1], (0, pad))
    cc = jnp.pad(covs_2d[:, 2], (0, pad), constant_values=1.0)
    dp = jnp.pad(depth_features[:, 0], (0, pad), constant_values=60.0)
    cr = jnp.pad(color_features[:, 0], (0, pad))
    cg = jnp.pad(color_features[:, 1], (0, pad))
    cbl = jnp.pad(color_features[:, 2], (0, pad))

    mesh = plsc.VectorSubcoreMesh(core_axis_name="c", subcore_axis_name="s")
    run = pl.kernel(
        _body,
        out_type=jax.ShapeDtypeStruct((3 * HW,), jnp.float32),
        mesh=mesh,
        compiler_params=pltpu.CompilerParams(needs_layout_passes=False),
        scratch_types=[
            *([pltpu.VMEM((CHUNK,), jnp.float32)] * 18),
            pltpu.VMEM_SHARED((QROWS,), jnp.float32),
            pltpu.VMEM_SHARED((QROWS,), jnp.float32),
            pltpu.VMEM_SHARED((QROWS,), jnp.float32),
            pltpu.VMEM_SHARED((QROWS,), jnp.float32),
            pltpu.VMEM((NB, 128), jnp.int32),
            pltpu.VMEM((CHUNK,), jnp.float32),
            pltpu.VMEM((CHUNK,), jnp.float32),
            pltpu.VMEM((CHUNK,), jnp.float32),
            pltpu.VMEM((CHUNK,), jnp.float32),
            pltpu.VMEM((NB, 128), jnp.int32),
            pltpu.VMEM((CHUNK,), jnp.float32),
            pltpu.VMEM((CHUNK,), jnp.float32),
            pltpu.VMEM((CHUNK,), jnp.float32),
            pltpu.VMEM((CHUNK,), jnp.float32),
            pltpu.VMEM((NCH,), jnp.float32),
            pltpu.VMEM((NCH,), jnp.float32),
            pltpu.VMEM((NCH,), jnp.float32),
            pltpu.VMEM((NCH,), jnp.float32),
            pltpu.VMEM((NCH,), jnp.float32),
            pltpu.VMEM((NCH,), jnp.float32),
            pltpu.VMEM((NCH,), jnp.float32),
            pltpu.VMEM((2048,), jnp.float32),
            pltpu.SemaphoreType.DMA,
            pltpu.SemaphoreType.DMA,
            pltpu.SemaphoreType.DMA,
            pltpu.SemaphoreType.DMA,
            pltpu.SemaphoreType.DMA,
            pltpu.SemaphoreType.DMA,
            pltpu.SemaphoreType.DMA,
        ],
    )
    out = run(xs, ys, ca, cb, cc, dp, cr, cg, cbl)
    return out.reshape(3, H, W)


# speedup vs baseline: 9.5244x; 1.1702x over previous
import jax
import jax.numpy as jnp
from jax import lax
from jax.experimental import pallas as pl
from jax.experimental.pallas import tpu as pltpu
from jax.experimental.pallas import tpu_sc as plsc

H = 1080
W = 1920
HW = H * W
EPS = 1e-8
P = 1_000_000

NUM_TILES = 16
CHUNK = 1024
NVEC = CHUNK // 16
NB = CHUNK // 128
PER_TILE = 65536
NCHUNKS = PER_TILE // CHUNK
NPAIR = NCHUNKS // 2
PPAD = PER_TILE * NUM_TILES

QSIZE = HW // 6
QROWS = 345728
ZROWS = QROWS // NUM_TILES
ZTAIL = ZROWS - 10 * 2048
NPIX = QSIZE // NUM_TILES
NCH = 1200

TWO_PI_INV = float(1.0 / (2.0 * 3.141592653589793))


def _rsqrt(x):
    i = plsc.bitcast(x, jnp.int32)
    i = jnp.int32(0x5F3759DF) - (i >> 1)
    y = plsc.bitcast(i, jnp.float32)
    for _ in range(3):
        y = y * (1.5 - 0.5 * x * y * y)
    return y


def _body(xs, ys, ca_, cb_, cc_, dp_, cr_, cg_, cbl_, out,
          iA0, iA1, iA2, iA3, iA4, iA5, iA6, iA7, iA8,
          iB0, iB1, iB2, iB3, iB4, iB5, iB6, iB7, iB8,
          accr, accg, accb, accw,
          idxA, vrA, vgA, vbA, vwA,
          idxB, vrB, vgB, vbB, vwB,
          rr, rg, rb, rw, obr, obg, obb, zbuf,
          semA, semB, semSA, semSB, semZ, semN, semO):
    core = lax.axis_index("c")
    tile = lax.axis_index("s")

    lanes = lax.iota(jnp.int32, 16)
    zf = jnp.zeros((16,), jnp.float32)

    def zinit(i, carry):
        zbuf[pl.ds(i * 16, 16)] = zf
        return carry

    lax.fori_loop(0, 128, zinit, jnp.int32(0))

    gtile0 = pl.multiple_of(tile * PER_TILE, 8)
    accs = (accr, accg, accb, accw)
    bufsA = (iA0, iA1, iA2, iA3, iA4, iA5, iA6, iA7, iA8)
    bufsB = (iB0, iB1, iB2, iB3, iB4, iB5, iB6, iB7, iB8)

    planes = (xs, ys, ca_, cb_, cc_, dp_, cr_, cg_, cbl_)

    def in_slices(g0):
        return tuple(pln.at[pl.ds(g0, CHUNK)] for pln in planes)

    def start_in(g0, bufs, sem):
        for src, dst in zip(in_slices(g0), bufs):
            pltpu.async_copy(src, dst, sem)

    def wait_in(g0, bufs, sem):
        for src, dst in zip(in_slices(g0), bufs):
            pltpu.make_async_copy(src, dst, sem).wait()

    def make_compute(bufs, idx2, vr, vg, vb, vw, qbase):
        bx, by, bca, bcb, bcc, bdp, bcr, bcg, bcbl = bufs

        def compute(i, cnt):
            sl16 = pl.ds(i * 16, 16)
            x = bx[sl16]
            y = by[sl16]
            ca = bca[sl16]
            cb = bcb[sl16]
            cc = bcc[sl16]
            dp = bdp[sl16]
            cr = bcr[sl16]
            cg = bcg[sl16]
            cbl = bcbl[sl16]

            px = (x * jnp.float32(W)).astype(jnp.int32)
            px = jnp.minimum(jnp.maximum(px, 0), W - 1)
            py = (y * jnp.float32(H)).astype(jnp.int32)
            py = jnp.minimum(jnp.maximum(py, 0), H - 1)
            pid = py * W + px

            det = jnp.maximum(ca * cc - cb * cb, jnp.float32(EPS))
            wgt = _rsqrt(det) * jnp.float32(TWO_PI_INV) * jnp.exp(-dp)

            local = pid - qbase
            inq = (local >= 0) & (local < QSIZE)
            inq_i = inq.astype(jnp.int32)
            pos = cnt + jnp.cumsum(inq_i) - 1
            plsc.store_scatter(idx2, [pos >> 7, pos & 127], local, mask=inq)
            plsc.store_scatter(vr, [pos], wgt * cr, mask=inq)
            plsc.store_scatter(vg, [pos], wgt * cg, mask=inq)
            plsc.store_scatter(vb, [pos], wgt * cbl, mask=inq)
            plsc.store_scatter(vw, [pos], wgt, mask=inq)
            return cnt + jnp.sum(inq_i)

        cnt = lax.fori_loop(0, NVEC, compute, jnp.int32(0))

        tb0 = lax.bitwise_and(cnt, jnp.int32(-128))
        tend = lax.bitwise_and(cnt + 127, jnp.int32(-128))
        for v in range(8):
            tpos = tb0 + v * 16 + lanes
            tmask = (tpos >= cnt) & (tpos < tend)
            plsc.store_scatter(idx2, [tpos >> 7, tpos & 127],
                               jnp.full((16,), QSIZE, jnp.int32), mask=tmask)
        return (cnt + 127) >> 7

    def sync_scatter(nbatch, idx2, vr, vg, vb, vw):
        def scat(j, carry):
            @pl.when(j < nbatch)
            def _():
                sl = pl.ds(j * 128, 128)
                idx = idx2.at[j]
                for v, acc in zip((vr, vg, vb, vw), accs):
                    pltpu.sync_copy(v.at[sl], acc.at[idx], add=True)
            return carry

        lax.fori_loop(0, NB, scat, jnp.int32(0))

    for p in range(3):
        qbase = (core * 3 + p) * QSIZE

        zrow0 = pl.multiple_of(tile * ZROWS, 8)

        def zfire(z, carry):
            zo = pl.multiple_of(zrow0 + z * 2048, 8)
            for acc in accs:
                pltpu.async_copy(zbuf, acc.at[pl.ds(zo, 2048)], semZ)
            return carry

        def zdrain(z, carry):
            zo = pl.multiple_of(zrow0 + z * 2048, 8)
            for acc in accs:
                pltpu.make_async_copy(zbuf, acc.at[pl.ds(zo, 2048)], semZ).wait()
            return carry

        lax.fori_loop(0, 10, zfire, jnp.int32(0))
        ztail0 = pl.multiple_of(zrow0 + 10 * 2048, 8)
        for acc in accs:
            pltpu.async_copy(zbuf.at[pl.ds(0, ZTAIL)],
                             acc.at[pl.ds(ztail0, ZTAIL)], semZ)
        lax.fori_loop(0, 10, zdrain, jnp.int32(0))
        for acc in accs:
            pltpu.make_async_copy(zbuf.at[pl.ds(0, ZTAIL)],
                                  acc.at[pl.ds(ztail0, ZTAIL)], semZ).wait()
        plsc.subcore_barrier()

        start_in(gtile0, bufsA, semA)

        def pair(i, nbB_prev):
            g0 = pl.multiple_of(gtile0 + (2 * i) * CHUNK, 8)
            g1 = pl.multiple_of(g0 + CHUNK, 8)

            wait_in(g0, bufsA, semA)
            start_in(g1, bufsB, semB)
            nbA = make_compute(bufsA, idxA, vrA, vgA, vbA, vwA, qbase)
            sync_scatter(nbA, idxA, vrA, vgA, vbA, vwA)

            wait_in(g1, bufsB, semB)

            @pl.when(i < NPAIR - 1)
            def _():
                g2 = pl.multiple_of(g1 + CHUNK, 8)
                start_in(g2, bufsA, semA)

            nbB = make_compute(bufsB, idxB, vrB, vgB, vbB, vwB, qbase)
            sync_scatter(nbB, idxB, vrB, vgB, vbB, vwB)
            return nbB_prev

        lax.fori_loop(0, NPAIR, pair, jnp.int32(0))

        plsc.subcore_barrier()

        prow0 = pl.multiple_of(tile * NPIX, 8)

        def norm_chunk(nc, ncarry):
            r0 = pl.multiple_of(prow0 + nc * NCH, 8)
            for acc, rbuf in zip(accs, (rr, rg, rb, rw)):
                pltpu.async_copy(acc.at[pl.ds(r0, NCH)], rbuf, semN)
            for acc, rbuf in zip(accs, (rr, rg, rb, rw)):
                pltpu.make_async_copy(acc.at[pl.ds(r0, NCH)], rbuf, semN).wait()

            gpix_prev = pl.multiple_of(qbase + r0 - NCH, 8)

            @pl.when(nc > 0)
            def _():
                for c, ob in enumerate((obr, obg, obb)):
                    pltpu.make_async_copy(
                        ob, out.at[pl.ds(c * HW + gpix_prev, NCH)], semO).wait()

            def norm(i, carry):
                sl = pl.ds(i * 16, 16)
                d = rw[sl] + jnp.float32(EPS)
                obr[sl] = rr[sl] / d
                obg[sl] = rg[sl] / d
                obb[sl] = rb[sl] / d
                return carry

            lax.fori_loop(0, NCH // 16, norm, jnp.int32(0))
            gpix = pl.multiple_of(qbase + r0, 8)
            for c, ob in enumerate((obr, obg, obb)):
                pltpu.async_copy(ob, out.at[pl.ds(c * HW + gpix, NCH)], semO)
            return ncarry

        lax.fori_loop(0, NPIX // NCH, norm_chunk, jnp.int32(0))
        gpix_last = pl.multiple_of(qbase + prow0 + NPIX - NCH, 8)
        for c, ob in enumerate((obr, obg, obb)):
            pltpu.make_async_copy(
                ob, out.at[pl.ds(c * HW + gpix_last, NCH)], semO).wait()

        plsc.subcore_barrier()


def kernel(means_2d, covs_2d, depth_features, color_features, height, width):
    pad = PPAD - P
    fill = jnp.linspace(0.0, 0.999, pad, dtype=jnp.float32)
    xs = jnp.concatenate([means_2d[:, 0], fill])
    ys = jnp.concatenate([means_2d[:, 1], fill])
    ca = jnp.pad(covs_2d[:, 0], (0, pad), constant_values=1.0)
    cb = jnp.pad(covs_2d[:, 1], (0, pad))
    cc = jnp.pad(covs_2d[:, 2], (0, pad), constant_values=1.0)
    dp = jnp.pad(depth_features[:, 0], (0, pad), constant_values=60.0)
    cr = jnp.pad(color_features[:, 0], (0, pad))
    cg = jnp.pad(color_features[:, 1], (0, pad))
    cbl = jnp.pad(color_features[:, 2], (0, pad))

    mesh = plsc.VectorSubcoreMesh(core_axis_name="c", subcore_axis_name="s")
    run = pl.kernel(
        _body,
        out_type=jax.ShapeDtypeStruct((3 * HW,), jnp.float32),
        mesh=mesh,
        compiler_params=pltpu.CompilerParams(needs_layout_passes=False),
        scratch_types=[
            *([pltpu.VMEM((CHUNK,), jnp.float32)] * 18),
            pltpu.VMEM_SHARED((QROWS,), jnp.float32),
            pltpu.VMEM_SHARED((QROWS,), jnp.float32),
            pltpu.VMEM_SHARED((QROWS,), jnp.float32),
            pltpu.VMEM_SHARED((QROWS,), jnp.float32),
            pltpu.VMEM((NB, 128), jnp.int32),
            pltpu.VMEM((CHUNK,), jnp.float32),
            pltpu.VMEM((CHUNK,), jnp.float32),
            pltpu.VMEM((CHUNK,), jnp.float32),
            pltpu.VMEM((CHUNK,), jnp.float32),
            pltpu.VMEM((NB, 128), jnp.int32),
            pltpu.VMEM((CHUNK,), jnp.float32),
            pltpu.VMEM((CHUNK,), jnp.float32),
            pltpu.VMEM((CHUNK,), jnp.float32),
            pltpu.VMEM((CHUNK,), jnp.float32),
            pltpu.VMEM((NCH,), jnp.float32),
            pltpu.VMEM((NCH,), jnp.float32),
            pltpu.VMEM((NCH,), jnp.float32),
            pltpu.VMEM((NCH,), jnp.float32),
            pltpu.VMEM((NCH,), jnp.float32),
            pltpu.VMEM((NCH,), jnp.float32),
            pltpu.VMEM((NCH,), jnp.float32),
            pltpu.VMEM((2048,), jnp.float32),
            pltpu.SemaphoreType.DMA,
            pltpu.SemaphoreType.DMA,
            pltpu.SemaphoreType.DMA,
            pltpu.SemaphoreType.DMA,
            pltpu.SemaphoreType.DMA,
            pltpu.SemaphoreType.DMA,
            pltpu.SemaphoreType.DMA,
        ],
    )
    out = run(xs, ys, ca, cb, cc, dp, cr, cg, cbl)
    return out.reshape(3, H, W)


# speedup vs baseline: 9.8052x; 1.0295x over previous
import jax
import jax.numpy as jnp
from jax import lax
from jax.experimental import pallas as pl
from jax.experimental.pallas import tpu as pltpu
from jax.experimental.pallas import tpu_sc as plsc

H = 1080
W = 1920
HW = H * W
EPS = 1e-8
P = 1_000_000

NUM_TILES = 16
CHUNK = 1024
NVEC = CHUNK // 16
NB = CHUNK // 128
PER_TILE = 65536
NCHUNKS = PER_TILE // CHUNK
NPAIR = NCHUNKS // 2
PPAD = PER_TILE * NUM_TILES

QSIZE = HW // 6
QROWS = 345728
ZROWS = QROWS // NUM_TILES
ZTAIL = ZROWS - 10 * 2048
NPIX = QSIZE // NUM_TILES
NCH = 1200

TWO_PI_INV = float(1.0 / (2.0 * 3.141592653589793))


def _rsqrt(x):
    i = plsc.bitcast(x, jnp.int32)
    i = jnp.int32(0x5F3759DF) - (i >> 1)
    y = plsc.bitcast(i, jnp.float32)
    for _ in range(3):
        y = y * (1.5 - 0.5 * x * y * y)
    return y


def _body(xs, ys, ca_, cb_, cc_, dp_, cr_, cg_, cbl_, out,
          iA0, iA1, iA2, iA3, iA4, iA5, iA6, iA7, iA8,
          iB0, iB1, iB2, iB3, iB4, iB5, iB6, iB7, iB8,
          accr, accg, accb, accw,
          idxA, vrA, vgA, vbA, vwA,
          idxB, vrB, vgB, vbB, vwB,
          rr, rg, rb, rw, obr, obg, obb, zbuf,
          semA, semB, semSA, semSB, semZ, semN, semO):
    core = lax.axis_index("c")
    tile = lax.axis_index("s")

    lanes = lax.iota(jnp.int32, 16)
    zf = jnp.zeros((16,), jnp.float32)

    def zinit(i, carry):
        zbuf[pl.ds(i * 16, 16)] = zf
        return carry

    lax.fori_loop(0, 128, zinit, jnp.int32(0))

    gtile0 = pl.multiple_of(tile * PER_TILE, 8)
    accs = (accr, accg, accb, accw)
    bufsA = (iA0, iA1, iA2, iA3, iA4, iA5, iA6, iA7, iA8)
    bufsB = (iB0, iB1, iB2, iB3, iB4, iB5, iB6, iB7, iB8)

    planes = (xs, ys, ca_, cb_, cc_, dp_, cr_, cg_, cbl_)

    def in_slices(g0):
        return tuple(pln.at[pl.ds(g0, CHUNK)] for pln in planes)

    def start_in(g0, bufs, sem):
        for src, dst in zip(in_slices(g0), bufs):
            pltpu.async_copy(src, dst, sem)

    def wait_in(g0, bufs, sem):
        for src, dst in zip(in_slices(g0), bufs):
            pltpu.make_async_copy(src, dst, sem).wait()

    def make_compute(bufs, idx2, vr, vg, vb, vw, qbase):
        bx, by, bca, bcb, bcc, bdp, bcr, bcg, bcbl = bufs

        def compute(i, cnt):
            sl16 = pl.ds(i * 16, 16)
            x = bx[sl16]
            y = by[sl16]
            ca = bca[sl16]
            cb = bcb[sl16]
            cc = bcc[sl16]
            dp = bdp[sl16]
            cr = bcr[sl16]
            cg = bcg[sl16]
            cbl = bcbl[sl16]

            px = (x * jnp.float32(W)).astype(jnp.int32)
            px = jnp.minimum(jnp.maximum(px, 0), W - 1)
            py = (y * jnp.float32(H)).astype(jnp.int32)
            py = jnp.minimum(jnp.maximum(py, 0), H - 1)
            pid = py * W + px

            det = jnp.maximum(ca * cc - cb * cb, jnp.float32(EPS))
            wgt = _rsqrt(det) * jnp.float32(TWO_PI_INV) * jnp.exp(-dp)

            local = pid - qbase
            inq = (local >= 0) & (local < QSIZE)
            inq_i = inq.astype(jnp.int32)
            pos = cnt + jnp.cumsum(inq_i) - 1
            plsc.store_scatter(idx2, [pos >> 7, pos & 127], local, mask=inq)
            plsc.store_scatter(vr, [pos], wgt * cr, mask=inq)
            plsc.store_scatter(vg, [pos], wgt * cg, mask=inq)
            plsc.store_scatter(vb, [pos], wgt * cbl, mask=inq)
            plsc.store_scatter(vw, [pos], wgt, mask=inq)
            return cnt + jnp.sum(inq_i)

        cnt = lax.fori_loop(0, NVEC, compute, jnp.int32(0))

        tb0 = lax.bitwise_and(cnt, jnp.int32(-128))
        tend = lax.bitwise_and(cnt + 127, jnp.int32(-128))
        for v in range(8):
            tpos = tb0 + v * 16 + lanes
            tmask = (tpos >= cnt) & (tpos < tend)
            plsc.store_scatter(idx2, [tpos >> 7, tpos & 127],
                               jnp.full((16,), QSIZE, jnp.int32), mask=tmask)
        return (cnt + 127) >> 7

    def sync_scatter(nbatch, idx2, vr, vg, vb, vw):
        def scat(j, carry):
            @pl.when(j < nbatch)
            def _():
                sl = pl.ds(j * 128, 128)
                idx = idx2.at[j]
                descs = [pltpu.async_copy(v.at[sl], acc.at[idx], semSA,
                                          add=True)
                         for v, acc in zip((vr, vg, vb, vw), accs)]
                for d in descs:
                    d.wait()
            return carry

        lax.fori_loop(0, NB, scat, jnp.int32(0))

    for p in range(3):
        qbase = (core * 3 + p) * QSIZE

        zrow0 = pl.multiple_of(tile * ZROWS, 8)

        def zfire(z, carry):
            zo = pl.multiple_of(zrow0 + z * 2048, 8)
            for acc in accs:
                pltpu.async_copy(zbuf, acc.at[pl.ds(zo, 2048)], semZ)
            return carry

        def zdrain(z, carry):
            zo = pl.multiple_of(zrow0 + z * 2048, 8)
            for acc in accs:
                pltpu.make_async_copy(zbuf, acc.at[pl.ds(zo, 2048)], semZ).wait()
            return carry

        lax.fori_loop(0, 10, zfire, jnp.int32(0))
        ztail0 = pl.multiple_of(zrow0 + 10 * 2048, 8)
        for acc in accs:
            pltpu.async_copy(zbuf.at[pl.ds(0, ZTAIL)],
                             acc.at[pl.ds(ztail0, ZTAIL)], semZ)
        lax.fori_loop(0, 10, zdrain, jnp.int32(0))
        for acc in accs:
            pltpu.make_async_copy(zbuf.at[pl.ds(0, ZTAIL)],
                                  acc.at[pl.ds(ztail0, ZTAIL)], semZ).wait()
        plsc.subcore_barrier()

        start_in(gtile0, bufsA, semA)

        def pair(i, nbB_prev):
            g0 = pl.multiple_of(gtile0 + (2 * i) * CHUNK, 8)
            g1 = pl.multiple_of(g0 + CHUNK, 8)

            wait_in(g0, bufsA, semA)
            start_in(g1, bufsB, semB)
            nbA = make_compute(bufsA, idxA, vrA, vgA, vbA, vwA, qbase)
            sync_scatter(nbA, idxA, vrA, vgA, vbA, vwA)

            wait_in(g1, bufsB, semB)

            @pl.when(i < NPAIR - 1)
            def _():
                g2 = pl.multiple_of(g1 + CHUNK, 8)
                start_in(g2, bufsA, semA)

            nbB = make_compute(bufsB, idxB, vrB, vgB, vbB, vwB, qbase)
            sync_scatter(nbB, idxB, vrB, vgB, vbB, vwB)
            return nbB_prev

        lax.fori_loop(0, NPAIR, pair, jnp.int32(0))

        plsc.subcore_barrier()

        prow0 = pl.multiple_of(tile * NPIX, 8)

        def norm_chunk(nc, ncarry):
            r0 = pl.multiple_of(prow0 + nc * NCH, 8)
            for acc, rbuf in zip(accs, (rr, rg, rb, rw)):
                pltpu.async_copy(acc.at[pl.ds(r0, NCH)], rbuf, semN)
            for acc, rbuf in zip(accs, (rr, rg, rb, rw)):
                pltpu.make_async_copy(acc.at[pl.ds(r0, NCH)], rbuf, semN).wait()

            gpix_prev = pl.multiple_of(qbase + r0 - NCH, 8)

            @pl.when(nc > 0)
            def _():
                for c, ob in enumerate((obr, obg, obb)):
                    pltpu.make_async_copy(
                        ob, out.at[pl.ds(c * HW + gpix_prev, NCH)], semO).wait()

            def norm(i, carry):
                sl = pl.ds(i * 16, 16)
                d = rw[sl] + jnp.float32(EPS)
                obr[sl] = rr[sl] / d
                obg[sl] = rg[sl] / d
                obb[sl] = rb[sl] / d
                return carry

            lax.fori_loop(0, NCH // 16, norm, jnp.int32(0))
            gpix = pl.multiple_of(qbase + r0, 8)
            for c, ob in enumerate((obr, obg, obb)):
                pltpu.async_copy(ob, out.at[pl.ds(c * HW + gpix, NCH)], semO)
            return ncarry

        lax.fori_loop(0, NPIX // NCH, norm_chunk, jnp.int32(0))
        gpix_last = pl.multiple_of(qbase + prow0 + NPIX - NCH, 8)
        for c, ob in enumerate((obr, obg, obb)):
            pltpu.make_async_copy(
                ob, out.at[pl.ds(c * HW + gpix_last, NCH)], semO).wait()

        plsc.subcore_barrier()


def kernel(means_2d, covs_2d, depth_features, color_features, height, width):
    pad = PPAD - P
    fill = jnp.linspace(0.0, 0.999, pad, dtype=jnp.float32)
    xs = jnp.concatenate([means_2d[:, 0], fill])
    ys = jnp.concatenate([means_2d[:, 1], fill])
    ca = jnp.pad(covs_2d[:, 0], (0, pad), constant_values=1.0)
    cb = jnp.pad(covs_2d[:, 1], (0, pad))
    cc = jnp.pad(covs_2d[:, 2], (0, pad), constant_values=1.0)
    dp = jnp.pad(depth_features[:, 0], (0, pad), constant_values=60.0)
    cr = jnp.pad(color_features[:, 0], (0, pad))
    cg = jnp.pad(color_features[:, 1], (0, pad))
    cbl = jnp.pad(color_features[:, 2], (0, pad))

    mesh = plsc.VectorSubcoreMesh(core_axis_name="c", subcore_axis_name="s")
    run = pl.kernel(
        _body,
        out_type=jax.ShapeDtypeStruct((3 * HW,), jnp.float32),
        mesh=mesh,
        compiler_params=pltpu.CompilerParams(needs_layout_passes=False),
        scratch_types=[
            *([pltpu.VMEM((CHUNK,), jnp.float32)] * 18),
            pltpu.VMEM_SHARED((QROWS,), jnp.float32),
            pltpu.VMEM_SHARED((QROWS,), jnp.float32),
            pltpu.VMEM_SHARED((QROWS,), jnp.float32),
            pltpu.VMEM_SHARED((QROWS,), jnp.float32),
            pltpu.VMEM((NB, 128), jnp.int32),
            pltpu.VMEM((CHUNK,), jnp.float32),
            pltpu.VMEM((CHUNK,), jnp.float32),
            pltpu.VMEM((CHUNK,), jnp.float32),
            pltpu.VMEM((CHUNK,), jnp.float32),
            pltpu.VMEM((NB, 128), jnp.int32),
            pltpu.VMEM((CHUNK,), jnp.float32),
            pltpu.VMEM((CHUNK,), jnp.float32),
            pltpu.VMEM((CHUNK,), jnp.float32),
            pltpu.VMEM((CHUNK,), jnp.float32),
            pltpu.VMEM((NCH,), jnp.float32),
            pltpu.VMEM((NCH,), jnp.float32),
            pltpu.VMEM((NCH,), jnp.float32),
            pltpu.VMEM((NCH,), jnp.float32),
            pltpu.VMEM((NCH,), jnp.float32),
            pltpu.VMEM((NCH,), jnp.float32),
            pltpu.VMEM((NCH,), jnp.float32),
            pltpu.VMEM((2048,), jnp.float32),
            pltpu.SemaphoreType.DMA,
            pltpu.SemaphoreType.DMA,
            pltpu.SemaphoreType.DMA,
            pltpu.SemaphoreType.DMA,
            pltpu.SemaphoreType.DMA,
            pltpu.SemaphoreType.DMA,
            pltpu.SemaphoreType.DMA,
        ],
    )
    out = run(xs, ys, ca, cb, cc, dp, cr, cg, cbl)
    return out.reshape(3, H, W)
